# baseline (speedup 1.0000x reference)
import functools

import jax
import jax.numpy as jnp
from jax import lax
from jax.experimental import pallas as pl
from jax.experimental.pallas import tpu as pltpu
from jax.experimental.pallas import tpu_sc as plsc

EMBED_DIM = 32
TOTAL_ROWS = 16384 * 26
NUM_WORKERS = 32
PER_WORKER = TOTAL_ROWS // NUM_WORKERS
ROWS_PER_STREAM = 128
STREAMS_PER_WORKER = PER_WORKER // ROWS_PER_STREAM
GROUP_STREAMS = 13
GROUP_ROWS = GROUP_STREAMS * ROWS_PER_STREAM
NUM_GROUPS = STREAMS_PER_WORKER // GROUP_STREAMS

_mesh = plsc.VectorSubcoreMesh(core_axis_name="c", subcore_axis_name="s")


@functools.partial(
    pl.kernel,
    mesh=_mesh,
    out_type=jax.ShapeDtypeStruct((TOTAL_ROWS, EMBED_DIM), jnp.bfloat16),
    scratch_types=[
        pltpu.VMEM((STREAMS_PER_WORKER, ROWS_PER_STREAM), jnp.int32),
        pltpu.VMEM((GROUP_ROWS, EMBED_DIM), jnp.bfloat16),
        pltpu.VMEM((GROUP_ROWS, EMBED_DIM), jnp.bfloat16),
        pltpu.SemaphoreType.DMA,
        pltpu.SemaphoreType.DMA,
        pltpu.SemaphoreType.DMA,
        pltpu.SemaphoreType.DMA,
    ],
    compiler_params=pltpu.CompilerParams(use_tc_tiling_on_sc=False),
)
def _gather_kernel(idx_hbm, table_hbm, out_hbm, idx_v, buf0, buf1,
                   gsem0, gsem1, osem0, osem1):
    wid = lax.axis_index("s") * 2 + lax.axis_index("c")
    base = wid * PER_WORKER
    bufs = (buf0, buf1)
    gsems = (gsem0, gsem1)
    osems = (osem0, osem1)

    pltpu.sync_copy(idx_hbm.at[wid], idx_v)

    def fire_group(g, buf, sem):
        for s in range(GROUP_STREAMS):
            pltpu.async_copy(
                table_hbm.at[idx_v.at[g * GROUP_STREAMS + s]],
                buf.at[pl.ds(s * ROWS_PER_STREAM, ROWS_PER_STREAM)],
                sem,
            )

    def drain_gathers(buf, sem):
        pltpu.make_async_copy(table_hbm.at[pl.ds(0, GROUP_ROWS)], buf, sem).wait()

    def fire_out(g, buf, sem):
        pltpu.async_copy(
            buf, out_hbm.at[pl.ds(base + g * GROUP_ROWS, GROUP_ROWS)], sem
        )

    def drain_out(buf, sem):
        pltpu.make_async_copy(buf, out_hbm.at[pl.ds(0, GROUP_ROWS)], sem).wait()

    fire_group(0, bufs[0], gsems[0])
    for g in range(NUM_GROUPS):
        p = g % 2
        q = 1 - p
        drain_gathers(bufs[p], gsems[p])
        if g + 1 < NUM_GROUPS:
            if g >= 1:
                drain_out(bufs[q], osems[q])
            fire_group(g + 1, bufs[q], gsems[q])
        fire_out(g, bufs[p], osems[p])
    drain_out(bufs[(NUM_GROUPS - 2) % 2], osems[(NUM_GROUPS - 2) % 2])
    drain_out(bufs[(NUM_GROUPS - 1) % 2], osems[(NUM_GROUPS - 1) % 2])


def kernel(input, table):
    idx = input.astype(jnp.int32).reshape(NUM_WORKERS, STREAMS_PER_WORKER, ROWS_PER_STREAM)
    out = _gather_kernel(idx, table.astype(jnp.bfloat16))
    return out.astype(jnp.float32).reshape(input.shape[0], input.shape[1], EMBED_DIM)


# speedup vs baseline: 1.3950x; 1.3950x over previous
import functools

import jax
import jax.numpy as jnp
from jax import lax
from jax.experimental import pallas as pl
from jax.experimental.pallas import tpu as pltpu
from jax.experimental.pallas import tpu_sc as plsc

EMBED_DIM = 32
TOTAL_ROWS = 16384 * 26
NUM_WORKERS = 32
PER_WORKER = TOTAL_ROWS // NUM_WORKERS
ROWS_PER_STREAM = 128
STREAMS_PER_WORKER = PER_WORKER // ROWS_PER_STREAM
GROUP_STREAMS = 13
GROUP_ROWS = GROUP_STREAMS * ROWS_PER_STREAM
NUM_GROUPS = STREAMS_PER_WORKER // GROUP_STREAMS

_mesh = plsc.VectorSubcoreMesh(core_axis_name="c", subcore_axis_name="s")


@functools.partial(
    pl.kernel,
    mesh=_mesh,
    out_type=jax.ShapeDtypeStruct((TOTAL_ROWS, EMBED_DIM), jnp.float32),
    scratch_types=[
        pltpu.VMEM((STREAMS_PER_WORKER, ROWS_PER_STREAM), jnp.int32),
        pltpu.VMEM((GROUP_ROWS, EMBED_DIM), jnp.float32),
        pltpu.VMEM((GROUP_ROWS, EMBED_DIM), jnp.float32),
        pltpu.SemaphoreType.DMA,
        pltpu.SemaphoreType.DMA,
        pltpu.SemaphoreType.DMA,
        pltpu.SemaphoreType.DMA,
    ],
    compiler_params=pltpu.CompilerParams(use_tc_tiling_on_sc=False),
)
def _gather_kernel(idx_hbm, table_hbm, out_hbm, idx_v, buf0, buf1,
                   gsem0, gsem1, osem0, osem1):
    wid = lax.axis_index("s") * 2 + lax.axis_index("c")
    base = wid * PER_WORKER
    bufs = (buf0, buf1)
    gsems = (gsem0, gsem1)
    osems = (osem0, osem1)

    pltpu.sync_copy(idx_hbm.at[wid], idx_v)

    def fire_group(g, buf, sem):
        for s in range(GROUP_STREAMS):
            pltpu.async_copy(
                table_hbm.at[idx_v.at[g * GROUP_STREAMS + s]],
                buf.at[pl.ds(s * ROWS_PER_STREAM, ROWS_PER_STREAM)],
                sem,
            )

    def drain_gathers(buf, sem):
        pltpu.make_async_copy(table_hbm.at[pl.ds(0, GROUP_ROWS)], buf, sem).wait()

    def fire_out(g, buf, sem):
        pltpu.async_copy(
            buf, out_hbm.at[pl.ds(base + g * GROUP_ROWS, GROUP_ROWS)], sem
        )

    def drain_out(buf, sem):
        pltpu.make_async_copy(buf, out_hbm.at[pl.ds(0, GROUP_ROWS)], sem).wait()

    fire_group(0, bufs[0], gsems[0])
    for g in range(NUM_GROUPS):
        p = g % 2
        q = 1 - p
        drain_gathers(bufs[p], gsems[p])
        if g + 1 < NUM_GROUPS:
            if g >= 1:
                drain_out(bufs[q], osems[q])
            fire_group(g + 1, bufs[q], gsems[q])
        fire_out(g, bufs[p], osems[p])
    drain_out(bufs[(NUM_GROUPS - 2) % 2], osems[(NUM_GROUPS - 2) % 2])
    drain_out(bufs[(NUM_GROUPS - 1) % 2], osems[(NUM_GROUPS - 1) % 2])


def kernel(input, table):
    idx = input.astype(jnp.int32).reshape(NUM_WORKERS, STREAMS_PER_WORKER, ROWS_PER_STREAM)
    out = _gather_kernel(idx, table)
    return out.reshape(input.shape[0], input.shape[1], EMBED_DIM)


# speedup vs baseline: 1.3995x; 1.0032x over previous
import functools

import jax
import jax.numpy as jnp
from jax import lax
from jax.experimental import pallas as pl
from jax.experimental.pallas import tpu as pltpu
from jax.experimental.pallas import tpu_sc as plsc

BATCH = 16384
N_FIELDS = 26
EMBED_DIM = 32
NUM_WORKERS = 32
ROWS_PER_WORKER = BATCH // NUM_WORKERS
GROUP = 64
NUM_GROUPS = ROWS_PER_WORKER // GROUP

_mesh = plsc.VectorSubcoreMesh(core_axis_name="c", subcore_axis_name="s")


@functools.partial(
    pl.kernel,
    mesh=_mesh,
    out_type=jax.ShapeDtypeStruct((BATCH, N_FIELDS, EMBED_DIM), jnp.float32),
    scratch_types=[
        pltpu.VMEM((ROWS_PER_WORKER, N_FIELDS), jnp.int32),
        pltpu.VMEM((GROUP, N_FIELDS, EMBED_DIM), jnp.float32),
        pltpu.VMEM((GROUP, N_FIELDS, EMBED_DIM), jnp.float32),
        pltpu.SemaphoreType.DMA,
        pltpu.SemaphoreType.DMA,
        pltpu.SemaphoreType.DMA,
        pltpu.SemaphoreType.DMA,
    ],
    compiler_params=pltpu.CompilerParams(use_tc_tiling_on_sc=False),
)
def _gather_kernel(idx_hbm, table_hbm, out_hbm, idx_v, buf0, buf1,
                   gsem0, gsem1, osem0, osem1):
    wid = lax.axis_index("s") * 2 + lax.axis_index("c")
    base = wid * ROWS_PER_WORKER
    bufs = (buf0, buf1)
    gsems = (gsem0, gsem1)
    osems = (osem0, osem1)

    pltpu.sync_copy(idx_hbm.at[pl.ds(base, ROWS_PER_WORKER)], idx_v)

    def fire_gather(g, buf, sem):
        def row_body(r, _):
            pltpu.async_copy(
                table_hbm.at[idx_v.at[g * GROUP + r]], buf.at[r], sem
            )
            return 0

        lax.fori_loop(0, GROUP, row_body, 0)

    def drain_gather(buf, sem):
        pltpu.make_async_copy(out_hbm.at[pl.ds(0, GROUP)], buf, sem).wait()

    def fire_out(g, buf, sem):
        pltpu.async_copy(buf, out_hbm.at[pl.ds(base + g * GROUP, GROUP)], sem)

    def drain_out(buf, sem):
        pltpu.make_async_copy(buf, out_hbm.at[pl.ds(0, GROUP)], sem).wait()

    fire_gather(0, bufs[0], gsems[0])
    for g in range(NUM_GROUPS):
        p = g % 2
        q = 1 - p
        drain_gather(bufs[p], gsems[p])
        if g + 1 < NUM_GROUPS:
            if g >= 1:
                drain_out(bufs[q], osems[q])
            fire_gather(g + 1, bufs[q], gsems[q])
        fire_out(g, bufs[p], osems[p])
    drain_out(bufs[NUM_GROUPS % 2], osems[NUM_GROUPS % 2])
    drain_out(bufs[(NUM_GROUPS - 1) % 2], osems[(NUM_GROUPS - 1) % 2])


def kernel(input, table):
    return _gather_kernel(input, table)
